# Initial kernel scaffold; baseline (speedup 1.0000x reference)
import jax
import jax.numpy as jnp
from jax.experimental import pallas as pl


def kernel(x, edge_index, W1, b1, W2, b2, Wlin, blin):
    raise NotImplementedError("write your pallas kernel here")


# speedup vs baseline: 13.9845x; 13.9845x over previous
import functools

import jax
import jax.numpy as jnp
from jax import lax
from jax.experimental import pallas as pl
from jax.experimental.pallas import tpu as pltpu, tpu_sc as plsc

NNODE = 10000
NEDGE = 320000
NF = 128
NCLS = 40

NC = 2
NS = 16
NW = NC * NS
CHUNK = 64
CPT = (-(-NEDGE // (NS * CHUNK)) + 7) // 8 * 8
EPAD = NS * CPT * CHUNK
EW = EPAD // NW
NP_ = 10240
H = NP_ // NC
TR = 128
AR = H + TR
RPT = H // NS
ZCH = AR // CHUNK
BM = 512
GRID = NP_ // BM

def _deg_body(dst_hbm, out_hbm, dst_v, hist_v):
    c = lax.axis_index("c")
    s = lax.axis_index("s")
    wid = c * NS + s

    def zbody(i, carry):
        hist_v[pl.ds(i * 16, 16)] = jnp.zeros((16,), jnp.float32)
        return carry

    lax.fori_loop(0, NP_ // 16, zbody, 0)
    pltpu.sync_copy(dst_hbm.at[pl.ds(wid * EW, EW)], dst_v)

    def gbody(g, carry):
        iv = dst_v[pl.ds(g * 16, 16)]
        plsc.addupdate_scatter(hist_v, [iv], jnp.ones((16,), jnp.float32))
        return carry

    lax.fori_loop(0, EW // 16, gbody, 0)
    pltpu.sync_copy(hist_v, out_hbm.at[wid])


@functools.cache
def _deg_call():
    mesh = plsc.VectorSubcoreMesh(core_axis_name="c", subcore_axis_name="s")
    return pl.kernel(
        _deg_body,
        compiler_params=pltpu.CompilerParams(needs_layout_passes=False),
        out_type=jax.ShapeDtypeStruct((NW, NP_), jnp.float32),
        mesh=mesh,
        scratch_types=[
            pltpu.VMEM((EW,), jnp.int32),
            pltpu.VMEM((NP_,), jnp.float32),
        ],
    )


NBUF = 2


def _scat_body(z_hbm, src_hbm, dst_hbm, zrow_hbm, out_hbm,
               src_v, db0, db1, r0, r1, acc,
               g0, g1, s0, s1, d0, d1):
    c = lax.axis_index("c")
    s = lax.axis_index("s")
    pltpu.sync_copy(zrow_hbm, r0)
    for t in range(-(-ZCH // NS)):
        k = s + NS * t

        @pl.when(k < ZCH)
        def _():
            pltpu.sync_copy(r0, acc.at[pl.ds(k * CHUNK, CHUNK)])

    pltpu.sync_copy(src_hbm.at[pl.ds(s * CPT, CPT)], src_v)
    ebase = c * EPAD + s * CPT * CHUNK
    plsc.subcore_barrier()
    rows = [r0, r1]
    dstb = [db0, db1]
    gsem = [g0, g1]
    ssem = [s0, s1]
    dsem = [d0, d1]
    for b in range(NBUF):
        pltpu.async_copy(dst_hbm.at[pl.ds(ebase + b * CHUNK, CHUNK)],
                         dstb[b], dsem[b])
        pltpu.async_copy(z_hbm.at[src_v.at[b]], rows[b], gsem[b])

    def round_body(g, carry):
        base = g * NBUF
        for b in range(NBUF):
            j = base + b
            pltpu.make_async_copy(
                dst_hbm.at[pl.ds(ebase + j * CHUNK, CHUNK)], dstb[b],
                dsem[b]).wait()
            pltpu.make_async_copy(z_hbm.at[src_v.at[j]], rows[b],
                                  gsem[b]).wait()
            pltpu.async_copy(rows[b], acc.at[dstb[b]], ssem[b], add=True)
        for b in range(NBUF):
            j = base + b
            pltpu.make_async_copy(rows[b], acc.at[dstb[b]], ssem[b]).wait()
            nj = j + NBUF

            @pl.when(nj < CPT)
            def _():
                pltpu.async_copy(
                    dst_hbm.at[pl.ds(ebase + nj * CHUNK, CHUNK)], dstb[b],
                    dsem[b])
                pltpu.async_copy(z_hbm.at[src_v.at[nj]], rows[b], gsem[b])

        return carry

    lax.fori_loop(0, CPT // NBUF, round_body, 0)
    plsc.subcore_barrier()
    pltpu.sync_copy(acc.at[pl.ds(s * RPT, RPT)],
                    out_hbm.at[pl.ds(c * H + s * RPT, RPT)])


@functools.cache
def _scat_call():
    mesh = plsc.VectorSubcoreMesh(core_axis_name="c", subcore_axis_name="s")
    return pl.kernel(
        _scat_body,
        out_type=jax.ShapeDtypeStruct((NP_, NF), jnp.float32),
        mesh=mesh,
        scratch_types=[
            pltpu.VMEM((CPT, CHUNK), jnp.int32),
            pltpu.VMEM((CHUNK,), jnp.int32),
            pltpu.VMEM((CHUNK,), jnp.int32),
            pltpu.VMEM((CHUNK, NF), jnp.float32),
            pltpu.VMEM((CHUNK, NF), jnp.float32),
            pltpu.VMEM_SHARED((AR, NF), jnp.float32),
            pltpu.SemaphoreType.DMA,
            pltpu.SemaphoreType.DMA,
            pltpu.SemaphoreType.DMA,
            pltpu.SemaphoreType.DMA,
            pltpu.SemaphoreType.DMA,
            pltpu.SemaphoreType.DMA,
        ],
    )


def _z1_body(x_ref, w_ref, degp_ref, z_ref):
    deg = jnp.sum(degp_ref[...], axis=0)[:, None] + 1.0
    dinv = lax.rsqrt(deg)
    z_ref[...] = jnp.dot(x_ref[...], w_ref[...],
                         preferred_element_type=jnp.float32) * dinv


_z1_call = pl.pallas_call(
    _z1_body,
    grid=(GRID,),
    in_specs=[
        pl.BlockSpec((BM, NF), lambda i: (i, 0)),
        pl.BlockSpec((NF, NF), lambda i: (0, 0)),
        pl.BlockSpec((NW, BM), lambda i: (0, i)),
    ],
    out_specs=pl.BlockSpec((BM, NF), lambda i: (i, 0)),
    out_shape=jax.ShapeDtypeStruct((NP_, NF), jnp.float32),
)


def _comb1_body(raw_ref, z1_ref, degp_ref, b1_ref, w2_ref, h1_ref, z2_ref):
    deg = jnp.sum(degp_ref[...], axis=0)[:, None] + 1.0
    dinv = lax.rsqrt(deg)
    h1 = jnp.maximum(
        dinv * (raw_ref[...] + z1_ref[...]) + b1_ref[...], 0.0)
    h1_ref[...] = h1
    z2_ref[...] = dinv * jnp.dot(h1, w2_ref[...],
                                 preferred_element_type=jnp.float32)


_comb1_call = pl.pallas_call(
    _comb1_body,
    grid=(GRID,),
    in_specs=[
        pl.BlockSpec((BM, NF), lambda i: (i, 0)),
        pl.BlockSpec((BM, NF), lambda i: (i, 0)),
        pl.BlockSpec((NW, BM), lambda i: (0, i)),
        pl.BlockSpec((1, NF), lambda i: (0, 0)),
        pl.BlockSpec((NF, NF), lambda i: (0, 0)),
    ],
    out_specs=[
        pl.BlockSpec((BM, NF), lambda i: (i, 0)),
        pl.BlockSpec((BM, NF), lambda i: (i, 0)),
    ],
    out_shape=[
        jax.ShapeDtypeStruct((NP_, NF), jnp.float32),
        jax.ShapeDtypeStruct((NP_, NF), jnp.float32),
    ],
)


def _comb2_body(raw_ref, z2_ref, degp_ref, b2_ref, h1_ref, wl_ref, bl_ref,
                o_ref):
    deg = jnp.sum(degp_ref[...], axis=0)[:, None] + 1.0
    dinv = lax.rsqrt(deg)
    h2 = jnp.maximum(
        dinv * (raw_ref[...] + z2_ref[...]) + b2_ref[...], 0.0)
    y = (jnp.dot(h1_ref[...], wl_ref[0:NF, :],
                 preferred_element_type=jnp.float32)
         + jnp.dot(h2, wl_ref[NF:2 * NF, :],
                   preferred_element_type=jnp.float32)
         + bl_ref[...])
    m = jnp.max(y, axis=1, keepdims=True)
    lse = jnp.log(jnp.sum(jnp.exp(y - m), axis=1, keepdims=True))
    o_ref[...] = y - m - lse


_comb2_call = pl.pallas_call(
    _comb2_body,
    grid=(GRID,),
    in_specs=[
        pl.BlockSpec((BM, NF), lambda i: (i, 0)),
        pl.BlockSpec((BM, NF), lambda i: (i, 0)),
        pl.BlockSpec((NW, BM), lambda i: (0, i)),
        pl.BlockSpec((1, NF), lambda i: (0, 0)),
        pl.BlockSpec((BM, NF), lambda i: (i, 0)),
        pl.BlockSpec((2 * NF, NF), lambda i: (0, 0)),
        pl.BlockSpec((1, NF), lambda i: (0, 0)),
    ],
    out_specs=pl.BlockSpec((BM, NF), lambda i: (i, 0)),
    out_shape=jax.ShapeDtypeStruct((NP_, NF), jnp.float32),
)


def kernel(x, edge_index, W1, b1, W2, b2, Wlin, blin):
    f32 = jnp.float32
    npad = EPAD - NEDGE
    pad_src = jnp.arange(npad, dtype=jnp.int32) % NNODE
    pad_dst = NNODE + jnp.arange(npad, dtype=jnp.int32) % (NP_ - NNODE)
    src2d = jnp.concatenate([edge_index[0], pad_src]).reshape(
        NS * CPT, CHUNK)
    dst = jnp.concatenate([edge_index[1], pad_dst])
    idx = jnp.arange(EPAD, dtype=jnp.int32)
    trash = H + (idx & (TR - 1))
    dsts = jnp.concatenate([
        jnp.where((dst >= c * H) & (dst < (c + 1) * H), dst - c * H, trash)
        for c in range(NC)])
    xp = jnp.zeros((NP_, NF), f32).at[:NNODE, :].set(x)
    zrow = jnp.zeros((CHUNK, NF), f32)

    degp = _deg_call()(dst)
    z1 = _z1_call(xp, W1, degp)
    raw1 = _scat_call()(z1, src2d, dsts, zrow)
    h1, z2 = _comb1_call(raw1, z1, degp, b1.reshape(1, NF), W2)
    raw2 = _scat_call()(z2, src2d, dsts, zrow)
    wl = jnp.zeros((2 * NF, NF), f32).at[:, :NCLS].set(Wlin)
    bl = jnp.full((1, NF), -1e30, f32).at[0, :NCLS].set(blin)
    out = _comb2_call(raw2, z2, degp, b2.reshape(1, NF), h1, wl, bl)
    return out[:NNODE, :NCLS]


# speedup vs baseline: 20.0766x; 1.4356x over previous
import functools

import jax
import jax.numpy as jnp
from jax import lax
from jax.experimental import pallas as pl
from jax.experimental.pallas import tpu as pltpu, tpu_sc as plsc

NNODE = 10000
NEDGE = 320000
NF = 128
NCLS = 40

NC = 2
NS = 16
NW = NC * NS
CHUNK = 128
CPT = (-(-NEDGE // (NS * CHUNK)) + 7) // 8 * 8
EPAD = NS * CPT * CHUNK
EW = EPAD // NW
NP_ = 10240
H = NP_ // NC
TR = 128
AR = H + TR
RPT = H // NS
ZCH = AR // CHUNK
BM = 512
GRID = NP_ // BM

def _deg_body(dst_hbm, out_hbm, dst_v, hist_v):
    c = lax.axis_index("c")
    s = lax.axis_index("s")
    wid = c * NS + s

    def zbody(i, carry):
        hist_v[pl.ds(i * 16, 16)] = jnp.zeros((16,), jnp.float32)
        return carry

    lax.fori_loop(0, NP_ // 16, zbody, 0)
    pltpu.sync_copy(dst_hbm.at[pl.ds(wid * EW, EW)], dst_v)

    def gbody(g, carry):
        iv = dst_v[pl.ds(g * 16, 16)]
        plsc.addupdate_scatter(hist_v, [iv], jnp.ones((16,), jnp.float32))
        return carry

    lax.fori_loop(0, EW // 16, gbody, 0)
    pltpu.sync_copy(hist_v, out_hbm.at[wid])


@functools.cache
def _deg_call():
    mesh = plsc.VectorSubcoreMesh(core_axis_name="c", subcore_axis_name="s")
    return pl.kernel(
        _deg_body,
        compiler_params=pltpu.CompilerParams(needs_layout_passes=False),
        out_type=jax.ShapeDtypeStruct((NW, NP_), jnp.float32),
        mesh=mesh,
        scratch_types=[
            pltpu.VMEM((EW,), jnp.int32),
            pltpu.VMEM((NP_,), jnp.float32),
        ],
    )


NBUF = 4


def _scat_body(z_hbm, src_hbm, dst_hbm, zrow_hbm, out_hbm,
               src_v, db0, db1, db2, db3, r0, r1, r2, r3, acc,
               g0, g1, g2, g3, s0, s1, s2, s3, d0, d1, d2, d3):
    c = lax.axis_index("c")
    s = lax.axis_index("s")
    pltpu.sync_copy(zrow_hbm, r0)
    for t in range(-(-ZCH // NS)):
        k = s + NS * t

        @pl.when(k < ZCH)
        def _():
            pltpu.sync_copy(r0, acc.at[pl.ds(k * CHUNK, CHUNK)])

    pltpu.sync_copy(src_hbm.at[pl.ds(s * CPT, CPT)], src_v)
    ebase = c * EPAD + s * CPT * CHUNK
    plsc.subcore_barrier()
    rows = [r0, r1, r2, r3]
    dstb = [db0, db1, db2, db3]
    gsem = [g0, g1, g2, g3]
    ssem = [s0, s1, s2, s3]
    dsem = [d0, d1, d2, d3]
    for b in range(NBUF):
        pltpu.async_copy(dst_hbm.at[pl.ds(ebase + b * CHUNK, CHUNK)],
                         dstb[b], dsem[b])
        pltpu.async_copy(z_hbm.at[src_v.at[b]], rows[b], gsem[b])

    def round_body(g, carry):
        base = g * NBUF
        for b in range(NBUF):
            j = base + b
            pltpu.make_async_copy(
                dst_hbm.at[pl.ds(ebase + j * CHUNK, CHUNK)], dstb[b],
                dsem[b]).wait()
            pltpu.make_async_copy(z_hbm.at[src_v.at[j]], rows[b],
                                  gsem[b]).wait()
            pltpu.async_copy(rows[b], acc.at[dstb[b]], ssem[b], add=True)
        for b in range(NBUF):
            j = base + b
            pltpu.make_async_copy(rows[b], acc.at[dstb[b]], ssem[b]).wait()
            nj = j + NBUF

            @pl.when(nj < CPT)
            def _():
                pltpu.async_copy(
                    dst_hbm.at[pl.ds(ebase + nj * CHUNK, CHUNK)], dstb[b],
                    dsem[b])
                pltpu.async_copy(z_hbm.at[src_v.at[nj]], rows[b], gsem[b])

        return carry

    lax.fori_loop(0, CPT // NBUF, round_body, 0)
    plsc.subcore_barrier()
    pltpu.sync_copy(acc.at[pl.ds(s * RPT, RPT)],
                    out_hbm.at[pl.ds(c * H + s * RPT, RPT)])


@functools.cache
def _scat_call():
    mesh = plsc.VectorSubcoreMesh(core_axis_name="c", subcore_axis_name="s")
    return pl.kernel(
        _scat_body,
        out_type=jax.ShapeDtypeStruct((NP_, NF), jnp.float32),
        mesh=mesh,
        scratch_types=(
            [pltpu.VMEM((CPT, CHUNK), jnp.int32)]
            + [pltpu.VMEM((CHUNK,), jnp.int32)] * NBUF
            + [pltpu.VMEM((CHUNK, NF), jnp.float32)] * NBUF
            + [pltpu.VMEM_SHARED((AR, NF), jnp.float32)]
            + [pltpu.SemaphoreType.DMA] * (3 * NBUF)
        ),
    )


def _z1_body(x_ref, w_ref, degp_ref, z_ref):
    deg = jnp.sum(degp_ref[...], axis=0)[:, None] + 1.0
    dinv = lax.rsqrt(deg)
    z_ref[...] = jnp.dot(x_ref[...], w_ref[...],
                         preferred_element_type=jnp.float32) * dinv


_z1_call = pl.pallas_call(
    _z1_body,
    grid=(GRID,),
    in_specs=[
        pl.BlockSpec((BM, NF), lambda i: (i, 0)),
        pl.BlockSpec((NF, NF), lambda i: (0, 0)),
        pl.BlockSpec((NW, BM), lambda i: (0, i)),
    ],
    out_specs=pl.BlockSpec((BM, NF), lambda i: (i, 0)),
    out_shape=jax.ShapeDtypeStruct((NP_, NF), jnp.float32),
)


def _comb1_body(raw_ref, z1_ref, degp_ref, b1_ref, w2_ref, h1_ref, z2_ref):
    deg = jnp.sum(degp_ref[...], axis=0)[:, None] + 1.0
    dinv = lax.rsqrt(deg)
    h1 = jnp.maximum(
        dinv * (raw_ref[...] + z1_ref[...]) + b1_ref[...], 0.0)
    h1_ref[...] = h1
    z2_ref[...] = dinv * jnp.dot(h1, w2_ref[...],
                                 preferred_element_type=jnp.float32)


_comb1_call = pl.pallas_call(
    _comb1_body,
    grid=(GRID,),
    in_specs=[
        pl.BlockSpec((BM, NF), lambda i: (i, 0)),
        pl.BlockSpec((BM, NF), lambda i: (i, 0)),
        pl.BlockSpec((NW, BM), lambda i: (0, i)),
        pl.BlockSpec((1, NF), lambda i: (0, 0)),
        pl.BlockSpec((NF, NF), lambda i: (0, 0)),
    ],
    out_specs=[
        pl.BlockSpec((BM, NF), lambda i: (i, 0)),
        pl.BlockSpec((BM, NF), lambda i: (i, 0)),
    ],
    out_shape=[
        jax.ShapeDtypeStruct((NP_, NF), jnp.float32),
        jax.ShapeDtypeStruct((NP_, NF), jnp.float32),
    ],
)


def _comb2_body(raw_ref, z2_ref, degp_ref, b2_ref, h1_ref, wl_ref, bl_ref,
                o_ref):
    deg = jnp.sum(degp_ref[...], axis=0)[:, None] + 1.0
    dinv = lax.rsqrt(deg)
    h2 = jnp.maximum(
        dinv * (raw_ref[...] + z2_ref[...]) + b2_ref[...], 0.0)
    y = (jnp.dot(h1_ref[...], wl_ref[0:NF, :],
                 preferred_element_type=jnp.float32)
         + jnp.dot(h2, wl_ref[NF:2 * NF, :],
                   preferred_element_type=jnp.float32)
         + bl_ref[...])
    m = jnp.max(y, axis=1, keepdims=True)
    lse = jnp.log(jnp.sum(jnp.exp(y - m), axis=1, keepdims=True))
    o_ref[...] = y - m - lse


_comb2_call = pl.pallas_call(
    _comb2_body,
    grid=(GRID,),
    in_specs=[
        pl.BlockSpec((BM, NF), lambda i: (i, 0)),
        pl.BlockSpec((BM, NF), lambda i: (i, 0)),
        pl.BlockSpec((NW, BM), lambda i: (0, i)),
        pl.BlockSpec((1, NF), lambda i: (0, 0)),
        pl.BlockSpec((BM, NF), lambda i: (i, 0)),
        pl.BlockSpec((2 * NF, NF), lambda i: (0, 0)),
        pl.BlockSpec((1, NF), lambda i: (0, 0)),
    ],
    out_specs=pl.BlockSpec((BM, NF), lambda i: (i, 0)),
    out_shape=jax.ShapeDtypeStruct((NP_, NF), jnp.float32),
)


def kernel(x, edge_index, W1, b1, W2, b2, Wlin, blin):
    f32 = jnp.float32
    npad = EPAD - NEDGE
    pad_src = jnp.arange(npad, dtype=jnp.int32) % NNODE
    pad_dst = NNODE + jnp.arange(npad, dtype=jnp.int32) % (NP_ - NNODE)
    src2d = jnp.concatenate([edge_index[0], pad_src]).reshape(
        NS * CPT, CHUNK)
    dst = jnp.concatenate([edge_index[1], pad_dst])
    idx = jnp.arange(EPAD, dtype=jnp.int32)
    trash = H + (idx & (TR - 1))
    dsts = jnp.concatenate([
        jnp.where((dst >= c * H) & (dst < (c + 1) * H), dst - c * H, trash)
        for c in range(NC)])
    xp = jnp.zeros((NP_, NF), f32).at[:NNODE, :].set(x)
    zrow = jnp.zeros((CHUNK, NF), f32)

    degp = _deg_call()(dst)
    z1 = _z1_call(xp, W1, degp)
    raw1 = _scat_call()(z1, src2d, dsts, zrow)
    h1, z2 = _comb1_call(raw1, z1, degp, b1.reshape(1, NF), W2)
    raw2 = _scat_call()(z2, src2d, dsts, zrow)
    wl = jnp.zeros((2 * NF, NF), f32).at[:, :NCLS].set(Wlin)
    bl = jnp.full((1, NF), -1e30, f32).at[0, :NCLS].set(blin)
    out = _comb2_call(raw2, z2, degp, b2.reshape(1, NF), h1, wl, bl)
    return out[:NNODE, :NCLS]


# speedup vs baseline: 25.3863x; 1.2645x over previous
import functools

import jax
import jax.numpy as jnp
from jax import lax
from jax.experimental import pallas as pl
from jax.experimental.pallas import tpu as pltpu, tpu_sc as plsc

NNODE = 10000
NEDGE = 320000
NF = 128
NCLS = 40

NC = 2
NS = 16
NW = NC * NS
CHUNK = 128
CPT = (-(-NEDGE // (NS * CHUNK)) + 7) // 8 * 8
EPAD = NS * CPT * CHUNK
EW = EPAD // NW
NP_ = 10240
CPE = CPT // NC
ZCH = NP_ // CHUNK
RPT = NP_ // NS
BM = 512
GRID = NP_ // BM

def _deg_body(dst_hbm, out_hbm, dst_v, hist_v):
    c = lax.axis_index("c")
    s = lax.axis_index("s")
    wid = c * NS + s

    def zbody(i, carry):
        hist_v[pl.ds(i * 16, 16)] = jnp.zeros((16,), jnp.float32)
        return carry

    lax.fori_loop(0, NP_ // 16, zbody, 0)
    pltpu.sync_copy(dst_hbm.at[pl.ds(wid * EW, EW)], dst_v)

    def gbody(g, carry):
        iv = dst_v[pl.ds(g * 16, 16)]
        plsc.addupdate_scatter(hist_v, [iv], jnp.ones((16,), jnp.float32))
        return carry

    lax.fori_loop(0, EW // 16, gbody, 0)
    pltpu.sync_copy(hist_v, out_hbm.at[wid])


@functools.cache
def _deg_call():
    mesh = plsc.VectorSubcoreMesh(core_axis_name="c", subcore_axis_name="s")
    return pl.kernel(
        _deg_body,
        compiler_params=pltpu.CompilerParams(needs_layout_passes=False),
        out_type=jax.ShapeDtypeStruct((NW, NP_), jnp.float32),
        mesh=mesh,
        scratch_types=[
            pltpu.VMEM((EW,), jnp.int32),
            pltpu.VMEM((NP_,), jnp.float32),
        ],
    )


NBUF = 2


def _scat_body(z_hbm, src_hbm, dst_hbm, zrow_hbm, out_hbm,
               src_v, db0, db1, r0, r1, acc,
               g0, g1, s0, s1, d0, d1):
    c = lax.axis_index("c")
    s = lax.axis_index("s")
    pltpu.sync_copy(zrow_hbm, r0)
    for t in range(-(-ZCH // NS)):
        k = s + NS * t

        @pl.when(k < ZCH)
        def _():
            pltpu.sync_copy(r0, acc.at[pl.ds(k * CHUNK, CHUNK)])

    tid = c * NS + s
    pltpu.sync_copy(src_hbm.at[pl.ds(tid * CPE, CPE)], src_v)
    ebase = tid * CPE * CHUNK
    plsc.subcore_barrier()
    rows = [r0, r1]
    dstb = [db0, db1]
    gsem = [g0, g1]
    ssem = [s0, s1]
    dsem = [d0, d1]
    for b in range(NBUF):
        pltpu.async_copy(dst_hbm.at[pl.ds(ebase + b * CHUNK, CHUNK)],
                         dstb[b], dsem[b])
        pltpu.async_copy(z_hbm.at[src_v.at[b]], rows[b], gsem[b])

    def round_body(g, carry):
        base = g * NBUF
        for b in range(NBUF):
            j = base + b
            pltpu.make_async_copy(
                dst_hbm.at[pl.ds(ebase + j * CHUNK, CHUNK)], dstb[b],
                dsem[b]).wait()
            pltpu.make_async_copy(z_hbm.at[src_v.at[j]], rows[b],
                                  gsem[b]).wait()
            pltpu.async_copy(rows[b], acc.at[dstb[b]], ssem[b], add=True)
        for b in range(NBUF):
            j = base + b
            pltpu.make_async_copy(rows[b], acc.at[dstb[b]], ssem[b]).wait()
            nj = j + NBUF

            @pl.when(nj < CPE)
            def _():
                pltpu.async_copy(
                    dst_hbm.at[pl.ds(ebase + nj * CHUNK, CHUNK)], dstb[b],
                    dsem[b])
                pltpu.async_copy(z_hbm.at[src_v.at[nj]], rows[b], gsem[b])

        return carry

    lax.fori_loop(0, CPE // NBUF, round_body, 0)
    plsc.subcore_barrier()
    pltpu.sync_copy(acc.at[pl.ds(s * RPT, RPT)],
                    out_hbm.at[pl.ds(c * NP_ + s * RPT, RPT)])


@functools.cache
def _scat_call():
    mesh = plsc.VectorSubcoreMesh(core_axis_name="c", subcore_axis_name="s")
    return pl.kernel(
        _scat_body,
        out_type=jax.ShapeDtypeStruct((NC * NP_, NF), jnp.float32),
        mesh=mesh,
        scratch_types=(
            [pltpu.VMEM((CPE, CHUNK), jnp.int32)]
            + [pltpu.VMEM((CHUNK,), jnp.int32)] * NBUF
            + [pltpu.VMEM((CHUNK, NF), jnp.float32)] * NBUF
            + [pltpu.VMEM_SHARED((NP_, NF), jnp.float32)]
            + [pltpu.SemaphoreType.DMA] * (3 * NBUF)
        ),
    )


def _z1_body(x_ref, w_ref, degp_ref, z_ref):
    deg = jnp.sum(degp_ref[...], axis=0)[:, None] + 1.0
    dinv = lax.rsqrt(deg)
    z_ref[...] = jnp.dot(x_ref[...], w_ref[...],
                         preferred_element_type=jnp.float32) * dinv


_z1_call = pl.pallas_call(
    _z1_body,
    grid=(GRID,),
    in_specs=[
        pl.BlockSpec((BM, NF), lambda i: (i, 0)),
        pl.BlockSpec((NF, NF), lambda i: (0, 0)),
        pl.BlockSpec((NW, BM), lambda i: (0, i)),
    ],
    out_specs=pl.BlockSpec((BM, NF), lambda i: (i, 0)),
    out_shape=jax.ShapeDtypeStruct((NP_, NF), jnp.float32),
)


def _comb1_body(rawa_ref, rawb_ref, z1_ref, degp_ref, b1_ref, w2_ref,
                h1_ref, z2_ref):
    deg = jnp.sum(degp_ref[...], axis=0)[:, None] + 1.0
    dinv = lax.rsqrt(deg)
    raw = rawa_ref[...] + rawb_ref[...]
    h1 = jnp.maximum(
        dinv * (raw + z1_ref[...]) + b1_ref[...], 0.0)
    h1_ref[...] = h1
    z2_ref[...] = dinv * jnp.dot(h1, w2_ref[...],
                                 preferred_element_type=jnp.float32)


_comb1_call = pl.pallas_call(
    _comb1_body,
    grid=(GRID,),
    in_specs=[
        pl.BlockSpec((BM, NF), lambda i: (i, 0)),
        pl.BlockSpec((BM, NF), lambda i: (i + GRID, 0)),
        pl.BlockSpec((BM, NF), lambda i: (i, 0)),
        pl.BlockSpec((NW, BM), lambda i: (0, i)),
        pl.BlockSpec((1, NF), lambda i: (0, 0)),
        pl.BlockSpec((NF, NF), lambda i: (0, 0)),
    ],
    out_specs=[
        pl.BlockSpec((BM, NF), lambda i: (i, 0)),
        pl.BlockSpec((BM, NF), lambda i: (i, 0)),
    ],
    out_shape=[
        jax.ShapeDtypeStruct((NP_, NF), jnp.float32),
        jax.ShapeDtypeStruct((NP_, NF), jnp.float32),
    ],
)


def _comb2_body(rawa_ref, rawb_ref, z2_ref, degp_ref, b2_ref, h1_ref,
                wl_ref, bl_ref, o_ref):
    deg = jnp.sum(degp_ref[...], axis=0)[:, None] + 1.0
    dinv = lax.rsqrt(deg)
    raw = rawa_ref[...] + rawb_ref[...]
    h2 = jnp.maximum(
        dinv * (raw + z2_ref[...]) + b2_ref[...], 0.0)
    y = (jnp.dot(h1_ref[...], wl_ref[0:NF, :],
                 preferred_element_type=jnp.float32)
         + jnp.dot(h2, wl_ref[NF:2 * NF, :],
                   preferred_element_type=jnp.float32)
         + bl_ref[...])
    m = jnp.max(y, axis=1, keepdims=True)
    lse = jnp.log(jnp.sum(jnp.exp(y - m), axis=1, keepdims=True))
    o_ref[...] = y - m - lse


_comb2_call = pl.pallas_call(
    _comb2_body,
    grid=(GRID,),
    in_specs=[
        pl.BlockSpec((BM, NF), lambda i: (i, 0)),
        pl.BlockSpec((BM, NF), lambda i: (i + GRID, 0)),
        pl.BlockSpec((BM, NF), lambda i: (i, 0)),
        pl.BlockSpec((NW, BM), lambda i: (0, i)),
        pl.BlockSpec((1, NF), lambda i: (0, 0)),
        pl.BlockSpec((BM, NF), lambda i: (i, 0)),
        pl.BlockSpec((2 * NF, NF), lambda i: (0, 0)),
        pl.BlockSpec((1, NF), lambda i: (0, 0)),
    ],
    out_specs=pl.BlockSpec((BM, NF), lambda i: (i, 0)),
    out_shape=jax.ShapeDtypeStruct((NP_, NF), jnp.float32),
)


def kernel(x, edge_index, W1, b1, W2, b2, Wlin, blin):
    f32 = jnp.float32
    npad = EPAD - NEDGE
    pad_src = jnp.arange(npad, dtype=jnp.int32) % NNODE
    pad_dst = NNODE + jnp.arange(npad, dtype=jnp.int32) % (NP_ - NNODE)
    src2d = jnp.concatenate([edge_index[0], pad_src]).reshape(
        NS * CPT, CHUNK)
    dst = jnp.concatenate([edge_index[1], pad_dst])
    xp = jnp.zeros((NP_, NF), f32).at[:NNODE, :].set(x)
    zrow = jnp.zeros((CHUNK, NF), f32)

    degp = _deg_call()(dst)
    z1 = _z1_call(xp, W1, degp)
    raw1 = _scat_call()(z1, src2d, dst, zrow)
    h1, z2 = _comb1_call(raw1, raw1, z1, degp, b1.reshape(1, NF), W2)
    raw2 = _scat_call()(z2, src2d, dst, zrow)
    wl = jnp.zeros((2 * NF, NF), f32).at[:, :NCLS].set(Wlin)
    bl = jnp.full((1, NF), -1e30, f32).at[0, :NCLS].set(blin)
    out = _comb2_call(raw2, raw2, z2, degp, b2.reshape(1, NF), h1, wl, bl)
    return out[:NNODE, :NCLS]


# speedup vs baseline: 30.1469x; 1.1875x over previous
import functools

import jax
import jax.numpy as jnp
from jax import lax
from jax.experimental import pallas as pl
from jax.experimental.pallas import tpu as pltpu, tpu_sc as plsc

NNODE = 10000
NEDGE = 320000
NF = 128
NCLS = 40

NC = 2
NS = 16
NW = NC * NS
CHUNK = 64
CPT = (-(-NEDGE // (NS * CHUNK)) + 7) // 8 * 8
EPAD = NS * CPT * CHUNK
EW = EPAD // NW
NP_ = 10240
CPE = CPT // NC
ZCH = NP_ // CHUNK
RPT = NP_ // NS
BM = 512
GRID = NP_ // BM

def _deg_body(dst_hbm, out_hbm, dst_v, hist_v):
    c = lax.axis_index("c")
    s = lax.axis_index("s")
    wid = c * NS + s

    def zbody(i, carry):
        hist_v[pl.ds(i * 16, 16)] = jnp.zeros((16,), jnp.float32)
        return carry

    lax.fori_loop(0, NP_ // 16, zbody, 0)
    pltpu.sync_copy(dst_hbm.at[pl.ds(wid * EW, EW)], dst_v)

    def gbody(g, carry):
        iv = dst_v[pl.ds(g * 16, 16)]
        plsc.addupdate_scatter(hist_v, [iv], jnp.ones((16,), jnp.float32))
        return carry

    lax.fori_loop(0, EW // 16, gbody, 0)
    pltpu.sync_copy(hist_v, out_hbm.at[wid])


@functools.cache
def _deg_call():
    mesh = plsc.VectorSubcoreMesh(core_axis_name="c", subcore_axis_name="s")
    return pl.kernel(
        _deg_body,
        compiler_params=pltpu.CompilerParams(needs_layout_passes=False),
        out_type=jax.ShapeDtypeStruct((NW, NP_), jnp.float32),
        mesh=mesh,
        scratch_types=[
            pltpu.VMEM((EW,), jnp.int32),
            pltpu.VMEM((NP_,), jnp.float32),
        ],
    )


NBUF = 4


def _scat_body(z_hbm, src_hbm, dst_hbm, zrow_hbm, out_hbm,
               src_v, db0, db1, db2, db3, r0, r1, r2, r3, acc,
               g0, g1, g2, g3, s0, s1, s2, s3, d0, d1, d2, d3):
    c = lax.axis_index("c")
    s = lax.axis_index("s")
    pltpu.sync_copy(zrow_hbm, r0)
    for t in range(-(-ZCH // NS)):
        k = s + NS * t

        @pl.when(k < ZCH)
        def _():
            pltpu.sync_copy(r0, acc.at[pl.ds(k * CHUNK, CHUNK)])

    tid = c * NS + s
    pltpu.sync_copy(src_hbm.at[pl.ds(tid * CPE * CHUNK, CPE * CHUNK)],
                    src_v)
    ebase = tid * CPE * CHUNK
    plsc.subcore_barrier()
    rows = [r0, r1, r2, r3]
    dstb = [db0, db1, db2, db3]
    gsem = [g0, g1, g2, g3]
    ssem = [s0, s1, s2, s3]
    dsem = [d0, d1, d2, d3]
    for b in range(NBUF):
        pltpu.async_copy(dst_hbm.at[pl.ds(ebase + b * CHUNK, CHUNK)],
                         dstb[b], dsem[b])
        pltpu.async_copy(z_hbm.at[src_v.at[pl.ds(b * CHUNK, CHUNK)]],
                         rows[b], gsem[b])

    def round_body(g, carry):
        base = g * NBUF
        for b in range(NBUF):
            j = base + b
            pltpu.make_async_copy(
                dst_hbm.at[pl.ds(ebase + j * CHUNK, CHUNK)], dstb[b],
                dsem[b]).wait()
            pltpu.make_async_copy(
                z_hbm.at[src_v.at[pl.ds(j * CHUNK, CHUNK)]], rows[b],
                gsem[b]).wait()
            pltpu.async_copy(rows[b], acc.at[dstb[b]], ssem[b], add=True)
        for b in range(NBUF):
            j = base + b
            pltpu.make_async_copy(rows[b], acc.at[dstb[b]], ssem[b]).wait()
            nj = j + NBUF

            @pl.when(nj < CPE)
            def _():
                pltpu.async_copy(
                    dst_hbm.at[pl.ds(ebase + nj * CHUNK, CHUNK)], dstb[b],
                    dsem[b])
                pltpu.async_copy(
                    z_hbm.at[src_v.at[pl.ds(nj * CHUNK, CHUNK)]],
                    rows[b], gsem[b])

        return carry

    lax.fori_loop(0, CPE // NBUF, round_body, 0)
    plsc.subcore_barrier()
    pltpu.sync_copy(acc.at[pl.ds(s * RPT, RPT)],
                    out_hbm.at[pl.ds(c * NP_ + s * RPT, RPT)])


@functools.cache
def _scat_call():
    mesh = plsc.VectorSubcoreMesh(core_axis_name="c", subcore_axis_name="s")
    return pl.kernel(
        _scat_body,
        out_type=jax.ShapeDtypeStruct((NC * NP_, NF), jnp.float32),
        mesh=mesh,
        scratch_types=(
            [pltpu.VMEM((CPE * CHUNK,), jnp.int32)]
            + [pltpu.VMEM((CHUNK,), jnp.int32)] * NBUF
            + [pltpu.VMEM((CHUNK, NF), jnp.float32)] * NBUF
            + [pltpu.VMEM_SHARED((NP_, NF), jnp.float32)]
            + [pltpu.SemaphoreType.DMA] * (3 * NBUF)
        ),
    )


def _z1_body(x_ref, w_ref, degp_ref, z_ref):
    deg = jnp.sum(degp_ref[...], axis=0)[:, None] + 1.0
    dinv = lax.rsqrt(deg)
    z_ref[...] = jnp.dot(x_ref[...], w_ref[...],
                         preferred_element_type=jnp.float32) * dinv


_z1_call = pl.pallas_call(
    _z1_body,
    grid=(GRID,),
    in_specs=[
        pl.BlockSpec((BM, NF), lambda i: (i, 0)),
        pl.BlockSpec((NF, NF), lambda i: (0, 0)),
        pl.BlockSpec((NW, BM), lambda i: (0, i)),
    ],
    out_specs=pl.BlockSpec((BM, NF), lambda i: (i, 0)),
    out_shape=jax.ShapeDtypeStruct((NP_, NF), jnp.float32),
)


def _comb1_body(rawa_ref, rawb_ref, z1_ref, degp_ref, b1_ref, w2_ref,
                h1_ref, z2_ref):
    deg = jnp.sum(degp_ref[...], axis=0)[:, None] + 1.0
    dinv = lax.rsqrt(deg)
    raw = rawa_ref[...] + rawb_ref[...]
    h1 = jnp.maximum(
        dinv * (raw + z1_ref[...]) + b1_ref[...], 0.0)
    h1_ref[...] = h1
    z2_ref[...] = dinv * jnp.dot(h1, w2_ref[...],
                                 preferred_element_type=jnp.float32)


_comb1_call = pl.pallas_call(
    _comb1_body,
    grid=(GRID,),
    in_specs=[
        pl.BlockSpec((BM, NF), lambda i: (i, 0)),
        pl.BlockSpec((BM, NF), lambda i: (i + GRID, 0)),
        pl.BlockSpec((BM, NF), lambda i: (i, 0)),
        pl.BlockSpec((NW, BM), lambda i: (0, i)),
        pl.BlockSpec((1, NF), lambda i: (0, 0)),
        pl.BlockSpec((NF, NF), lambda i: (0, 0)),
    ],
    out_specs=[
        pl.BlockSpec((BM, NF), lambda i: (i, 0)),
        pl.BlockSpec((BM, NF), lambda i: (i, 0)),
    ],
    out_shape=[
        jax.ShapeDtypeStruct((NP_, NF), jnp.float32),
        jax.ShapeDtypeStruct((NP_, NF), jnp.float32),
    ],
)


def _comb2_body(rawa_ref, rawb_ref, z2_ref, degp_ref, b2_ref, h1_ref,
                wl_ref, bl_ref, o_ref):
    deg = jnp.sum(degp_ref[...], axis=0)[:, None] + 1.0
    dinv = lax.rsqrt(deg)
    raw = rawa_ref[...] + rawb_ref[...]
    h2 = jnp.maximum(
        dinv * (raw + z2_ref[...]) + b2_ref[...], 0.0)
    y = (jnp.dot(h1_ref[...], wl_ref[0:NF, :],
                 preferred_element_type=jnp.float32)
         + jnp.dot(h2, wl_ref[NF:2 * NF, :],
                   preferred_element_type=jnp.float32)
         + bl_ref[...])
    m = jnp.max(y, axis=1, keepdims=True)
    lse = jnp.log(jnp.sum(jnp.exp(y - m), axis=1, keepdims=True))
    o_ref[...] = y - m - lse


_comb2_call = pl.pallas_call(
    _comb2_body,
    grid=(GRID,),
    in_specs=[
        pl.BlockSpec((BM, NF), lambda i: (i, 0)),
        pl.BlockSpec((BM, NF), lambda i: (i + GRID, 0)),
        pl.BlockSpec((BM, NF), lambda i: (i, 0)),
        pl.BlockSpec((NW, BM), lambda i: (0, i)),
        pl.BlockSpec((1, NF), lambda i: (0, 0)),
        pl.BlockSpec((BM, NF), lambda i: (i, 0)),
        pl.BlockSpec((2 * NF, NF), lambda i: (0, 0)),
        pl.BlockSpec((1, NF), lambda i: (0, 0)),
    ],
    out_specs=pl.BlockSpec((BM, NF), lambda i: (i, 0)),
    out_shape=jax.ShapeDtypeStruct((NP_, NF), jnp.float32),
)


def kernel(x, edge_index, W1, b1, W2, b2, Wlin, blin):
    f32 = jnp.float32
    npad = EPAD - NEDGE
    pad_src = jnp.arange(npad, dtype=jnp.int32) % NNODE
    pad_dst = NNODE + jnp.arange(npad, dtype=jnp.int32) % (NP_ - NNODE)
    src = jnp.concatenate([edge_index[0], pad_src])
    dst = jnp.concatenate([edge_index[1], pad_dst])
    xp = jnp.zeros((NP_, NF), f32).at[:NNODE, :].set(x)
    zrow = jnp.zeros((CHUNK, NF), f32)

    degp = _deg_call()(dst)
    z1 = _z1_call(xp, W1, degp)
    raw1 = _scat_call()(z1, src, dst, zrow)
    h1, z2 = _comb1_call(raw1, raw1, z1, degp, b1.reshape(1, NF), W2)
    raw2 = _scat_call()(z2, src, dst, zrow)
    wl = jnp.zeros((2 * NF, NF), f32).at[:, :NCLS].set(Wlin)
    bl = jnp.full((1, NF), -1e30, f32).at[0, :NCLS].set(blin)
    out = _comb2_call(raw2, raw2, z2, degp, b2.reshape(1, NF), h1, wl, bl)
    return out[:NNODE, :NCLS]
